# baseline (speedup 1.0000x reference)
import jax
import jax.numpy as jnp
from jax import lax
from jax.experimental import pallas as pl
from jax.experimental.pallas import tpu as pltpu
from jax.experimental.pallas import tpu_sc as plsc

BATCH, SEQ, DIM = 4, 4096, 2048
ROWS = BATCH * SEQ
NC, NS = 2, 16
NW = NC * NS
SC_ROWS = 8192
ROWS_PER_W = SC_ROWS // NW
CHUNK = 2
NCHUNK = ROWS_PER_W // CHUNK
LANES = 16


def _permute_chunk(in_v, out_v, perm_v):
    rvecs = [jnp.full((LANES,), r, jnp.int32) for r in range(CHUNK)]

    @plsc.parallel_loop(0, DIM // LANES, unroll=4)
    def _(j):
        cidx = perm_v[pl.ds(j * LANES, LANES)]
        jo = j * LANES
        for r in range(CHUNK):
            val = plsc.load_gather(in_v, [rvecs[r], cidx])
            out_v[r, pl.ds(jo, LANES)] = val


NBUF = 8


def _shuffle_body(x_hbm, perm_hbm, out_hbm, perm_v, *scratch):
    in_bufs = scratch[:NBUF]
    out_bufs = scratch[NBUF:2 * NBUF]
    si = scratch[2 * NBUF:3 * NBUF]
    so = scratch[3 * NBUF:4 * NBUF]

    wid = lax.axis_index("s") * NC + lax.axis_index("c")
    base = wid * ROWS_PER_W
    pltpu.sync_copy(perm_hbm, perm_v)

    def in_dma(c, k):
        return pltpu.make_async_copy(
            x_hbm.at[pl.ds(base + c * CHUNK, CHUNK)], in_bufs[k], si[k])

    def out_dma(c, k):
        return pltpu.make_async_copy(
            out_bufs[k], out_hbm.at[pl.ds(base + c * CHUNK, CHUNK)], so[k])

    for k in range(NBUF):
        in_dma(k, k).start()

    def ring_body(cc, carry):
        c0 = NBUF * cc
        for k in range(NBUF):
            c = c0 + k
            in_dma(c, k).wait()

            @pl.when(cc > 0)
            def _():
                out_dma(c - NBUF, k).wait()

            _permute_chunk(in_bufs[k], out_bufs[k], perm_v)

            @pl.when(cc < (NCHUNK // NBUF - 1))
            def _():
                in_dma(c + NBUF, k).start()

            out_dma(c, k).start()
        return carry

    lax.fori_loop(0, NCHUNK // NBUF, ring_body, 0)
    for k in range(NBUF):
        out_dma(NCHUNK - NBUF + k, k).wait()


@jax.jit
def _shuffle(x2, perm):
    mesh = plsc.VectorSubcoreMesh(core_axis_name="c", subcore_axis_name="s")
    f = pl.kernel(
        _shuffle_body,
        out_type=jax.ShapeDtypeStruct((ROWS, DIM), jnp.float32),
        mesh=mesh,
        scratch_types=(
            [pltpu.VMEM((DIM,), jnp.int32)]
            + [pltpu.VMEM((CHUNK, DIM), jnp.float32)] * (2 * NBUF)
            + [pltpu.SemaphoreType.DMA] * (2 * NBUF)
        ),
        compiler_params=pltpu.CompilerParams(needs_layout_passes=False),
    )
    return f(x2, perm)


TC_BLK = 1024


def _tc_body(perm_ref, x_ref, o_ref, p_scratch):
    @pl.when(pl.program_id(0) == 0)
    def _():
        row_iota = lax.broadcasted_iota(jnp.int32, (DIM, DIM), 0)
        p_scratch[...] = (row_iota == perm_ref[...]).astype(jnp.float32)

    o_ref[...] = jnp.dot(x_ref[...], p_scratch[...],
                         preferred_element_type=jnp.float32)


def _tc_shuffle(x2, perm2, r1, nrows):
    grid = nrows // TC_BLK
    return pl.pallas_call(
        _tc_body,
        grid=(grid,),
        in_specs=[
            pl.BlockSpec((1, DIM), lambda i: (0, 0)),
            pl.BlockSpec((TC_BLK, DIM), lambda i: (r1 // TC_BLK + i, 0)),
        ],
        out_specs=pl.BlockSpec((TC_BLK, DIM), lambda i: (i, 0)),
        out_shape=jax.ShapeDtypeStruct((nrows, DIM), jnp.float32),
        scratch_shapes=[pltpu.VMEM((DIM, DIM), jnp.float32)],
    )(perm2, x2)


def kernel(x, permutation):
    x2 = x.reshape(ROWS, DIM)
    perm = permutation.astype(jnp.int32)
    sc_full = _shuffle(x2, perm)
    tc_part = _tc_shuffle(x2, perm[None, :], SC_ROWS, ROWS - SC_ROWS)
    out = lax.dynamic_update_slice(sc_full, tc_part, (SC_ROWS, 0))
    return out.reshape(BATCH, SEQ, DIM)


# speedup vs baseline: 1.3333x; 1.3333x over previous
import jax
import jax.numpy as jnp
from jax import lax
from jax.experimental import pallas as pl
from jax.experimental.pallas import tpu as pltpu
from jax.experimental.pallas import tpu_sc as plsc

BATCH, SEQ, DIM = 4, 4096, 2048
ROWS = BATCH * SEQ
NC, NS = 2, 16
NW = NC * NS
ROWS_PER_W = ROWS // NW
CHUNK = 4
NCHUNK = ROWS_PER_W // CHUNK
LANES = 16


def _permute_chunk(in_v, out_v, perm_v):
    rvecs = [jnp.full((LANES,), r, jnp.int32) for r in range(CHUNK)]

    @plsc.parallel_loop(0, DIM // LANES, unroll=4)
    def _(j):
        cidx = perm_v[pl.ds(j * LANES, LANES)]
        jo = j * LANES
        for r in range(CHUNK):
            val = plsc.load_gather(in_v, [rvecs[r], cidx])
            out_v[r, pl.ds(jo, LANES)] = val


NBUF = 4


def _shuffle_body(x_hbm, perm_hbm, out_hbm, perm_v, *scratch):
    in_bufs = scratch[:NBUF]
    out_bufs = scratch[NBUF:2 * NBUF]
    si = scratch[2 * NBUF:3 * NBUF]
    so = scratch[3 * NBUF:4 * NBUF]

    wid = lax.axis_index("s") * NC + lax.axis_index("c")
    base = wid * ROWS_PER_W
    pltpu.sync_copy(perm_hbm, perm_v)

    def in_dma(c, k):
        return pltpu.make_async_copy(
            x_hbm.at[pl.ds(base + c * CHUNK, CHUNK)], in_bufs[k], si[k])

    def out_dma(c, k):
        return pltpu.make_async_copy(
            out_bufs[k], out_hbm.at[pl.ds(base + c * CHUNK, CHUNK)], so[k])

    for k in range(NBUF):
        in_dma(k, k).start()

    def ring_body(cc, carry):
        c0 = NBUF * cc
        for k in range(NBUF):
            c = c0 + k
            in_dma(c, k).wait()

            @pl.when(cc > 0)
            def _():
                out_dma(c - NBUF, k).wait()

            _permute_chunk(in_bufs[k], out_bufs[k], perm_v)

            @pl.when(cc < (NCHUNK // NBUF - 1))
            def _():
                in_dma(c + NBUF, k).start()

            out_dma(c, k).start()
        return carry

    lax.fori_loop(0, NCHUNK // NBUF, ring_body, 0)
    for k in range(NBUF):
        out_dma(NCHUNK - NBUF + k, k).wait()


@jax.jit
def _shuffle(x2, perm):
    mesh = plsc.VectorSubcoreMesh(core_axis_name="c", subcore_axis_name="s")
    f = pl.kernel(
        _shuffle_body,
        out_type=jax.ShapeDtypeStruct((ROWS, DIM), jnp.float32),
        mesh=mesh,
        scratch_types=(
            [pltpu.VMEM((DIM,), jnp.int32)]
            + [pltpu.VMEM((CHUNK, DIM), jnp.float32)] * (2 * NBUF)
            + [pltpu.SemaphoreType.DMA] * (2 * NBUF)
        ),
        compiler_params=pltpu.CompilerParams(needs_layout_passes=False),
    )
    return f(x2, perm)


def kernel(x, permutation):
    x2 = x.reshape(ROWS, DIM)
    perm = permutation.astype(jnp.int32)
    out = _shuffle(x2, perm)
    return out.reshape(BATCH, SEQ, DIM)
